# Initial kernel scaffold; baseline (speedup 1.0000x reference)
import functools

import jax
import jax.numpy as jnp
from jax import lax
from jax.experimental import pallas as pl
from jax.experimental.pallas import tpu as pltpu
from jax.experimental.pallas import tpu_sc as plsc

N = 10000
E = 320000
D = 128
NC = 2
NS = 16
NW = NC * NS
EPW = E // NW
C = 125
NCH = EPW // C
RPT = N // NS
DEGW = 16


def _sc_aggregate(feat, srcw, dstw):
    mesh = plsc.VectorSubcoreMesh(core_axis_name="c", subcore_axis_name="s")

    @functools.partial(
        pl.kernel,
        out_type=(
            jax.ShapeDtypeStruct((NC, N, D), jnp.float32),
            jax.ShapeDtypeStruct((NC, N, DEGW), jnp.float32),
        ),
        mesh=mesh,
        scratch_types=[
            pltpu.VMEM((C, D), jnp.float32),
            pltpu.VMEM((C, D), jnp.float32),
            pltpu.VMEM((NCH, C), jnp.int32),
            pltpu.VMEM((NCH, C), jnp.int32),
            pltpu.VMEM((C, DEGW), jnp.float32),
            pltpu.SemaphoreType.DMA,
            pltpu.SemaphoreType.DMA,
            pltpu.VMEM_SHARED((N, D), jnp.float32),
            pltpu.VMEM_SHARED((N, DEGW), jnp.float32),
        ],
    )
    def sc_agg(feat_hbm, srcw_hbm, dstw_hbm, part_hbm, degw_hbm,
               rows0, rows1, src_v, dst_v, ones_v, sem0, sem1, acc, degs):
        c = lax.axis_index("c")
        s = lax.axis_index("s")
        wid = c * NS + s
        rows = (rows0, rows1)
        sems = (sem0, sem1)
        zero16 = jnp.zeros((16,), jnp.float32)
        one16 = jnp.ones((16,), jnp.float32)

        def zr(i, carry):
            for k in range(D // 16):
                rows0[i, pl.ds(k * 16, 16)] = zero16
            ones_v[i, :] = zero16
            return carry
        lax.fori_loop(0, C, zr, 0)

        base = s * RPT
        for k in range(RPT // C):
            pltpu.sync_copy(rows0, acc.at[pl.ds(base + k * C, C)])
            pltpu.sync_copy(ones_v, degs.at[pl.ds(base + k * C, C)])

        def so(i, carry):
            ones_v[i, :] = one16
            return carry
        lax.fori_loop(0, C, so, 0)

        pltpu.sync_copy(srcw_hbm.at[wid], src_v)
        pltpu.sync_copy(dstw_hbm.at[wid], dst_v)

        plsc.subcore_barrier()

        pltpu.async_copy(feat_hbm.at[src_v.at[0]], rows0, sem0)
        pltpu.async_copy(feat_hbm.at[src_v.at[1]], rows1, sem1)

        def chunk(jj, b):
            pltpu.make_async_copy(
                feat_hbm.at[pl.ds(0, C)], rows[b], sems[b]).wait()
            pltpu.sync_copy(rows[b], acc.at[dst_v.at[jj]], add=True)
            pltpu.sync_copy(ones_v, degs.at[dst_v.at[jj]], add=True)

            @pl.when(jj + 2 < NCH)
            def _():
                pltpu.async_copy(
                    feat_hbm.at[src_v.at[jj + 2]], rows[b], sems[b])

        def outer(i, carry):
            chunk(i * 2, 0)
            chunk(i * 2 + 1, 1)
            return carry
        lax.fori_loop(0, NCH // 2, outer, 0)

        plsc.subcore_barrier()

        for k in range(RPT // C):
            off = base + k * C
            pltpu.sync_copy(acc.at[pl.ds(off, C)],
                            part_hbm.at[c, pl.ds(off, C)])
            pltpu.sync_copy(degs.at[pl.ds(off, C)],
                            degw_hbm.at[c, pl.ds(off, C)])

    return sc_agg(feat, srcw, dstw)


def _tc_combine(feat, part, degw, wn_t, ws_t, b2):
    R = 1000

    def body(f_ref, p_ref, d_ref, wn_ref, ws_ref, b_ref, o_ref):
        x = f_ref[...]
        accm = p_ref[0] + p_ref[1]
        deg = d_ref[0, :, 0:1] + d_ref[1, :, 0:1]
        scale = 1.0 / jnp.maximum(deg, 1.0)
        hn = jnp.dot(accm * scale, wn_ref[...],
                     preferred_element_type=jnp.float32)
        hs = jnp.dot(x, ws_ref[...], preferred_element_type=jnp.float32)
        o_ref[...] = hs + b_ref[...] + hn

    return pl.pallas_call(
        body,
        grid=(N // R,),
        in_specs=[
            pl.BlockSpec((R, D), lambda i: (i, 0)),
            pl.BlockSpec((NC, R, D), lambda i: (0, i, 0)),
            pl.BlockSpec((NC, R, DEGW), lambda i: (0, i, 0)),
            pl.BlockSpec((D, D), lambda i: (0, 0)),
            pl.BlockSpec((D, D), lambda i: (0, 0)),
            pl.BlockSpec((1, D), lambda i: (0, 0)),
        ],
        out_specs=pl.BlockSpec((R, D), lambda i: (i, 0)),
        out_shape=jax.ShapeDtypeStruct((N, D), jnp.float32),
    )(feat, part, degw, wn_t, ws_t, b2)


def kernel(feat, edge_index, W_neigh, W_self, b_self):
    src = edge_index[0].astype(jnp.int32)
    dst = edge_index[1].astype(jnp.int32)
    srcw = src.reshape(NW, NCH, C)
    dstw = dst.reshape(NW, NCH, C)
    part, degw = _sc_aggregate(feat, srcw, dstw)
    return _tc_combine(feat, part, degw, W_neigh.T, W_self.T,
                       b_self.reshape(1, D))


# speedup vs baseline: 9.7216x; 9.7216x over previous
import functools

import jax
import jax.numpy as jnp
from jax import lax
from jax.experimental import pallas as pl
from jax.experimental.pallas import tpu as pltpu
from jax.experimental.pallas import tpu_sc as plsc

N = 10000
E = 320000
D = 128
DH = D // 2
NC = 2
NS = 16
EPT = E // NS
C = 80
NCH = EPT // C
NU = N // C
DEGW = 16


def _sc_aggregate(flo, fhi, srcw, dstw):
    mesh = plsc.VectorSubcoreMesh(core_axis_name="c", subcore_axis_name="s")

    @functools.partial(
        pl.kernel,
        out_type=(
            jax.ShapeDtypeStruct((NC, N, DH), jnp.float32),
            jax.ShapeDtypeStruct((NC, N, DEGW), jnp.float32),
        ),
        mesh=mesh,
        compiler_params=pltpu.CompilerParams(use_tc_tiling_on_sc=False),
        scratch_types=[
            pltpu.VMEM((C, DH), jnp.float32),
            pltpu.VMEM((C, DH), jnp.float32),
            pltpu.VMEM((NCH, C), jnp.int32),
            pltpu.VMEM((NCH, C), jnp.int32),
            pltpu.VMEM((C, DEGW), jnp.float32),
            pltpu.SemaphoreType.DMA,
            pltpu.SemaphoreType.DMA,
            pltpu.VMEM_SHARED((N, DH), jnp.float32),
            pltpu.VMEM_SHARED((N, DEGW), jnp.float32),
        ],
    )
    def sc_agg(flo_hbm, fhi_hbm, srcw_hbm, dstw_hbm, part_hbm, degw_hbm,
               rows0, rows1, src_v, dst_v, ones_v, sem0, sem1, acc, degs):
        c = lax.axis_index("c")
        s = lax.axis_index("s")
        rows = (rows0, rows1)
        sems = (sem0, sem1)
        zero16 = jnp.zeros((16,), jnp.float32)
        one16 = jnp.ones((16,), jnp.float32)

        def zr(i, carry):
            for k in range(DH // 16):
                rows0[i, pl.ds(k * 16, 16)] = zero16
            ones_v[i, :] = zero16
            return carry
        lax.fori_loop(0, C, zr, 0)

        for k in range(pl.cdiv(NU, NS)):
            u = s + k * NS

            @pl.when(u < NU)
            def _():
                pltpu.sync_copy(rows0, acc.at[pl.ds(u * C, C)])
                pltpu.sync_copy(ones_v, degs.at[pl.ds(u * C, C)])

        def so(i, carry):
            ones_v[i, :] = one16
            return carry
        lax.fori_loop(0, C, so, 0)

        pltpu.sync_copy(srcw_hbm.at[s], src_v)
        pltpu.sync_copy(dstw_hbm.at[s], dst_v)

        plsc.subcore_barrier()

        def fire(jj, b):
            @pl.when(c == 0)
            def _():
                pltpu.async_copy(flo_hbm.at[src_v.at[jj]], rows[b], sems[b])

            @pl.when(c == 1)
            def _():
                pltpu.async_copy(fhi_hbm.at[src_v.at[jj]], rows[b], sems[b])

        fire(0, 0)
        fire(1, 1)

        def chunk(jj, b):
            pltpu.make_async_copy(
                flo_hbm.at[pl.ds(0, C)], rows[b], sems[b]).wait()
            pltpu.sync_copy(rows[b], acc.at[dst_v.at[jj]], add=True)
            deg_mine = jnp.where(c == 0, jj < NCH // 2, jj >= NCH // 2)

            @pl.when(deg_mine)
            def _():
                pltpu.sync_copy(ones_v, degs.at[dst_v.at[jj]], add=True)

            @pl.when(jj + 2 < NCH)
            def _():
                fire(jj + 2, b)

        def outer(i, carry):
            chunk(i * 2, 0)
            chunk(i * 2 + 1, 1)
            return carry
        lax.fori_loop(0, NCH // 2, outer, 0)

        plsc.subcore_barrier()

        for k in range(pl.cdiv(NU, NS)):
            u = s + k * NS

            @pl.when(u < NU)
            def _():
                pltpu.sync_copy(acc.at[pl.ds(u * C, C)],
                                part_hbm.at[c, pl.ds(u * C, C)])
                pltpu.sync_copy(degs.at[pl.ds(u * C, C)],
                                degw_hbm.at[c, pl.ds(u * C, C)])

    return sc_agg(flo, fhi, srcw, dstw)


def _tc_combine(feat, part, degw, wn_t, ws_t, b2):
    R = 1000

    def body(f_ref, p_ref, d_ref, wn_ref, ws_ref, b_ref, o_ref):
        x = f_ref[...]
        accm = jnp.concatenate([p_ref[0], p_ref[1]], axis=1)
        deg = d_ref[0, :, 0:1] + d_ref[1, :, 0:1]
        scale = 1.0 / jnp.maximum(deg, 1.0)
        hn = jnp.dot(accm * scale, wn_ref[...],
                     preferred_element_type=jnp.float32)
        hs = jnp.dot(x, ws_ref[...], preferred_element_type=jnp.float32)
        o_ref[...] = hs + b_ref[...] + hn

    return pl.pallas_call(
        body,
        grid=(N // R,),
        in_specs=[
            pl.BlockSpec((R, D), lambda i: (i, 0)),
            pl.BlockSpec((NC, R, DH), lambda i: (0, i, 0)),
            pl.BlockSpec((NC, R, DEGW), lambda i: (0, i, 0)),
            pl.BlockSpec((D, D), lambda i: (0, 0)),
            pl.BlockSpec((D, D), lambda i: (0, 0)),
            pl.BlockSpec((1, D), lambda i: (0, 0)),
        ],
        out_specs=pl.BlockSpec((R, D), lambda i: (i, 0)),
        out_shape=jax.ShapeDtypeStruct((N, D), jnp.float32),
    )(feat, part, degw, wn_t, ws_t, b2)


def kernel(feat, edge_index, W_neigh, W_self, b_self):
    src = edge_index[0].astype(jnp.int32)
    dst = edge_index[1].astype(jnp.int32)
    srcw = src.reshape(NS, NCH, C)
    dstw = dst.reshape(NS, NCH, C)
    flo = feat[:, :DH]
    fhi = feat[:, DH:]
    part, degw = _sc_aggregate(flo, fhi, srcw, dstw)
    return _tc_combine(feat, part, degw, W_neigh.T, W_self.T,
                       b_self.reshape(1, D))


# speedup vs baseline: 11.1936x; 1.1514x over previous
import functools

import jax
import jax.numpy as jnp
from jax import lax
from jax.experimental import pallas as pl
from jax.experimental.pallas import tpu as pltpu
from jax.experimental.pallas import tpu_sc as plsc

N = 10000
E = 320000
D = 128
DH = D // 2
NC = 2
NS = 16
EPT = E // NS
C = 80
NCH = EPT // C
NU = N // C
DEGW = 16
NB = 4


def _sc_aggregate(ftab, srcw, dstw):
    mesh = plsc.VectorSubcoreMesh(core_axis_name="c", subcore_axis_name="s")

    @functools.partial(
        pl.kernel,
        out_type=(
            jax.ShapeDtypeStruct((NC, N, DH), jnp.float32),
            jax.ShapeDtypeStruct((NC, N, DEGW), jnp.float32),
        ),
        mesh=mesh,
        compiler_params=pltpu.CompilerParams(use_tc_tiling_on_sc=False),
        scratch_types=[
            [pltpu.VMEM((C, DH), jnp.float32) for _ in range(NB)],
            pltpu.VMEM((NCH, C), jnp.int32),
            pltpu.VMEM((NCH, C), jnp.int32),
            pltpu.VMEM((C, DEGW), jnp.float32),
            [pltpu.SemaphoreType.DMA for _ in range(NB)],
            [pltpu.SemaphoreType.DMA for _ in range(NB)],
            pltpu.VMEM_SHARED((N, DH), jnp.float32),
            pltpu.VMEM_SHARED((N, DEGW), jnp.float32),
        ],
    )
    def sc_agg(ftab_hbm, srcw_hbm, dstw_hbm, part_hbm, degw_hbm,
               rows, src_v, dst_v, ones_v, sem_g, sem_s, acc, degs):
        c = lax.axis_index("c")
        s = lax.axis_index("s")
        zero16 = jnp.zeros((16,), jnp.float32)
        one16 = jnp.ones((16,), jnp.float32)

        def zr(i, carry):
            for k in range(DH // 16):
                rows[0][i, pl.ds(k * 16, 16)] = zero16
            ones_v[i, :] = zero16
            return carry
        lax.fori_loop(0, C, zr, 0)

        for k in range(pl.cdiv(NU, NS)):
            u = s + k * NS

            @pl.when(u < NU)
            def _():
                pltpu.sync_copy(rows[0], acc.at[pl.ds(u * C, C)])
                pltpu.sync_copy(ones_v, degs.at[pl.ds(u * C, C)])

        def so(i, carry):
            ones_v[i, :] = one16
            return carry
        lax.fori_loop(0, C, so, 0)

        pltpu.sync_copy(srcw_hbm.at[s], src_v)
        pltpu.sync_copy(dstw_hbm.at[s], dst_v)

        def tx(i, carry):
            for k in range(C // 16):
                v = src_v[i, pl.ds(k * 16, 16)]
                src_v[i, pl.ds(k * 16, 16)] = v + v + c
            return carry
        lax.fori_loop(0, NCH, tx, 0)

        plsc.subcore_barrier()

        def fire(jj, b):
            pltpu.async_copy(ftab_hbm.at[src_v.at[jj]], rows[b], sem_g[b])

        def wait_g(b):
            pltpu.make_async_copy(
                ftab_hbm.at[pl.ds(0, C)], rows[b], sem_g[b]).wait()

        def wait_s(b):
            pltpu.make_async_copy(
                rows[b], acc.at[pl.ds(0, C)], sem_s[b]).wait()

        fire(0, 0)
        fire(1, 1)

        def chunk(jj, b):
            wait_g(b)
            pltpu.async_copy(rows[b], acc.at[dst_v.at[jj]], sem_s[b],
                             add=True)
            deg_mine = jnp.where(c == 0, jj < NCH // 2, jj >= NCH // 2)

            @pl.when(deg_mine)
            def _():
                pltpu.sync_copy(ones_v, degs.at[dst_v.at[jj]], add=True)

            bb = (b + 2) % NB

            @pl.when(jj + 2 < NCH)
            def _():
                @pl.when(jj >= 2)
                def _():
                    wait_s(bb)
                fire(jj + 2, bb)

        def outer(i, carry):
            for u in range(NB):
                chunk(i * NB + u, u)
            return carry
        lax.fori_loop(0, NCH // NB, outer, 0)
        for jj in range(NCH - NCH % NB, NCH):
            chunk(jj, jj % NB)

        for b in range(NB):
            wait_s(b)

        plsc.subcore_barrier()

        for k in range(pl.cdiv(NU, NS)):
            u = s + k * NS

            @pl.when(u < NU)
            def _():
                pltpu.sync_copy(acc.at[pl.ds(u * C, C)],
                                part_hbm.at[c, pl.ds(u * C, C)])
                pltpu.sync_copy(degs.at[pl.ds(u * C, C)],
                                degw_hbm.at[c, pl.ds(u * C, C)])

    return sc_agg(ftab, srcw, dstw)


def _tc_combine(feat, part, degw, wn_t, ws_t, b2):
    R = 1000

    def body(f_ref, p_ref, d_ref, wn_ref, ws_ref, b_ref, o_ref):
        x = f_ref[...]
        deg = d_ref[0, :, 0:1] + d_ref[1, :, 0:1]
        scale = 1.0 / jnp.maximum(deg, 1.0)
        hn = jnp.dot(p_ref[0] * scale, wn_ref[0:DH, :],
                     preferred_element_type=jnp.float32)
        hn += jnp.dot(p_ref[1] * scale, wn_ref[DH:D, :],
                      preferred_element_type=jnp.float32)
        hs = jnp.dot(x, ws_ref[...], preferred_element_type=jnp.float32)
        o_ref[...] = hs + b_ref[...] + hn

    return pl.pallas_call(
        body,
        grid=(N // R,),
        in_specs=[
            pl.BlockSpec((R, D), lambda i: (i, 0)),
            pl.BlockSpec((NC, R, DH), lambda i: (0, i, 0)),
            pl.BlockSpec((NC, R, DEGW), lambda i: (0, i, 0)),
            pl.BlockSpec((D, D), lambda i: (0, 0)),
            pl.BlockSpec((D, D), lambda i: (0, 0)),
            pl.BlockSpec((1, D), lambda i: (0, 0)),
        ],
        out_specs=pl.BlockSpec((R, D), lambda i: (i, 0)),
        out_shape=jax.ShapeDtypeStruct((N, D), jnp.float32),
    )(feat, part, degw, wn_t, ws_t, b2)


def kernel(feat, edge_index, W_neigh, W_self, b_self):
    src = edge_index[0].astype(jnp.int32)
    dst = edge_index[1].astype(jnp.int32)
    srcw = src.reshape(NS, NCH, C)
    dstw = dst.reshape(NS, NCH, C)
    ftab = feat.reshape(N * NC, DH)
    part, degw = _sc_aggregate(ftab, srcw, dstw)
    return _tc_combine(feat, part, degw, W_neigh.T, W_self.T,
                       b_self.reshape(1, D))


# speedup vs baseline: 13.6817x; 1.2223x over previous
import functools

import jax
import jax.numpy as jnp
from jax import lax
from jax.experimental import pallas as pl
from jax.experimental.pallas import tpu as pltpu
from jax.experimental.pallas import tpu_sc as plsc

N = 10000
E = 320000
D = 128
DH = D // 2
NC = 2
NS = 16
EPT = E // NS
C = 160
NCH = EPT // C
U = 80
NU = N // U
DEGW = 16
NB = 3


def _sc_aggregate(ftab, srcw, dstw):
    mesh = plsc.VectorSubcoreMesh(core_axis_name="c", subcore_axis_name="s")

    @functools.partial(
        pl.kernel,
        out_type=(
            jax.ShapeDtypeStruct((NC, N, DH), jnp.float32),
            jax.ShapeDtypeStruct((NC, N, DEGW), jnp.float32),
        ),
        mesh=mesh,
        compiler_params=pltpu.CompilerParams(use_tc_tiling_on_sc=False),
        scratch_types=[
            [pltpu.VMEM((C, DH), jnp.float32) for _ in range(NB)],
            pltpu.VMEM((NCH, C), jnp.int32),
            pltpu.VMEM((NCH, C), jnp.int32),
            pltpu.VMEM((C, DEGW), jnp.float32),
            [pltpu.SemaphoreType.DMA for _ in range(NB)],
            [pltpu.SemaphoreType.DMA for _ in range(NB)],
            pltpu.VMEM_SHARED((N, DH), jnp.float32),
            pltpu.VMEM_SHARED((N, DEGW), jnp.float32),
        ],
    )
    def sc_agg(ftab_hbm, srcw_hbm, dstw_hbm, part_hbm, degw_hbm,
               rows, src_v, dst_v, ones_v, sem_g, sem_s, acc, degs):
        c = lax.axis_index("c")
        s = lax.axis_index("s")
        zero16 = jnp.zeros((16,), jnp.float32)
        one16 = jnp.ones((16,), jnp.float32)

        def zr(i, carry):
            for k in range(DH // 16):
                rows[0][i, pl.ds(k * 16, 16)] = zero16
            ones_v[i, :] = zero16
            return carry
        lax.fori_loop(0, C, zr, 0)

        for k in range(pl.cdiv(NU, NS)):
            u = s + k * NS

            @pl.when(u < NU)
            def _():
                pltpu.sync_copy(rows[0].at[pl.ds(0, U)],
                                acc.at[pl.ds(u * U, U)])
                pltpu.sync_copy(ones_v.at[pl.ds(0, U)],
                                degs.at[pl.ds(u * U, U)])

        def so(i, carry):
            ones_v[i, :] = one16
            return carry
        lax.fori_loop(0, C, so, 0)

        pltpu.sync_copy(srcw_hbm.at[s], src_v)
        pltpu.sync_copy(dstw_hbm.at[s], dst_v)

        def tx(i, carry):
            for k in range(C // 16):
                v = src_v[i, pl.ds(k * 16, 16)]
                src_v[i, pl.ds(k * 16, 16)] = v + v + c
            return carry
        lax.fori_loop(0, NCH, tx, 0)

        plsc.subcore_barrier()

        def fire(jj, b):
            pltpu.async_copy(ftab_hbm.at[src_v.at[jj]], rows[b], sem_g[b])

        def wait_g(b):
            pltpu.make_async_copy(
                ftab_hbm.at[pl.ds(0, C)], rows[b], sem_g[b]).wait()

        def wait_s(b):
            pltpu.make_async_copy(
                rows[b], acc.at[pl.ds(0, C)], sem_s[b]).wait()

        fire(0, 0)
        fire(1, 1)

        def chunk(jj, b):
            wait_g(b)
            pltpu.async_copy(rows[b], acc.at[dst_v.at[jj]], sem_s[b],
                             add=True)
            deg_mine = jnp.where(c == 0, jj < NCH // 2, jj >= NCH // 2)

            @pl.when(deg_mine)
            def _():
                pltpu.sync_copy(ones_v, degs.at[dst_v.at[jj]], add=True)

            bb = (b + 2) % NB

            @pl.when(jj + 2 < NCH)
            def _():
                @pl.when(jj >= NB - 2)
                def _():
                    wait_s(bb)
                fire(jj + 2, bb)

        def outer(i, carry):
            for u in range(NB):
                chunk(i * NB + u, u)
            return carry
        lax.fori_loop(0, NCH // NB, outer, 0)
        for jj in range(NCH - NCH % NB, NCH):
            chunk(jj, jj % NB)

        for b in range(NB):
            wait_s(b)

        plsc.subcore_barrier()

        for k in range(pl.cdiv(NU, NS)):
            u = s + k * NS

            @pl.when(u < NU)
            def _():
                pltpu.sync_copy(acc.at[pl.ds(u * U, U)],
                                part_hbm.at[c, pl.ds(u * U, U)])
                pltpu.sync_copy(degs.at[pl.ds(u * U, U)],
                                degw_hbm.at[c, pl.ds(u * U, U)])

    return sc_agg(ftab, srcw, dstw)


def _tc_combine(feat, part, degw, wn_t, ws_t, b2):
    R = 1000

    def body(f_ref, p_ref, d_ref, wn_ref, ws_ref, b_ref, o_ref):
        x = f_ref[...]
        deg = d_ref[0, :, 0:1] + d_ref[1, :, 0:1]
        scale = 1.0 / jnp.maximum(deg, 1.0)
        hn = jnp.dot(p_ref[0] * scale, wn_ref[0:DH, :],
                     preferred_element_type=jnp.float32)
        hn += jnp.dot(p_ref[1] * scale, wn_ref[DH:D, :],
                      preferred_element_type=jnp.float32)
        hs = jnp.dot(x, ws_ref[...], preferred_element_type=jnp.float32)
        o_ref[...] = hs + b_ref[...] + hn

    return pl.pallas_call(
        body,
        grid=(N // R,),
        in_specs=[
            pl.BlockSpec((R, D), lambda i: (i, 0)),
            pl.BlockSpec((NC, R, DH), lambda i: (0, i, 0)),
            pl.BlockSpec((NC, R, DEGW), lambda i: (0, i, 0)),
            pl.BlockSpec((D, D), lambda i: (0, 0)),
            pl.BlockSpec((D, D), lambda i: (0, 0)),
            pl.BlockSpec((1, D), lambda i: (0, 0)),
        ],
        out_specs=pl.BlockSpec((R, D), lambda i: (i, 0)),
        out_shape=jax.ShapeDtypeStruct((N, D), jnp.float32),
    )(feat, part, degw, wn_t, ws_t, b2)


def kernel(feat, edge_index, W_neigh, W_self, b_self):
    src = edge_index[0].astype(jnp.int32)
    dst = edge_index[1].astype(jnp.int32)
    srcw = src.reshape(NS, NCH, C)
    dstw = dst.reshape(NS, NCH, C)
    ftab = feat.reshape(N * NC, DH)
    part, degw = _sc_aggregate(ftab, srcw, dstw)
    return _tc_combine(feat, part, degw, W_neigh.T, W_self.T,
                       b_self.reshape(1, D))
